# Initial kernel scaffold; baseline (speedup 1.0000x reference)
import jax
import jax.numpy as jnp
from jax.experimental import pallas as pl


def kernel(e3_idx, table, W1, b1, W2, b2):
    raise NotImplementedError("write your pallas kernel here")


# speedup vs baseline: 1.8218x; 1.8218x over previous
import functools

import jax
import jax.numpy as jnp
from jax import lax
from jax.experimental import pallas as pl
from jax.experimental.pallas import tpu as pltpu
from jax.experimental.pallas import tpu_sc as plsc

NUM_E3 = 12
E3_DIM = 64
HID = 32
BATCH = 16384

_NC = 2
_NS = 16
_L = 16
_NW = _NC * _NS
_BPW = BATCH // _NW


def _mlp_body(tp_ref, w1_ref, b1_ref, w2t_ref, b2_ref, logits_ref, score_ref):
    t = tp_ref[...]
    h = jnp.maximum(
        jnp.dot(t, w1_ref[...], preferred_element_type=jnp.float32) + b1_ref[...],
        0.0,
    )
    lg = jnp.sum(h * w2t_ref[...], axis=1, keepdims=True) + b2_ref[...]
    logits_ref[...] = lg
    score_ref[...] = jax.nn.sigmoid(lg)


@functools.lru_cache(maxsize=None)
def _mlp_call():
    return pl.pallas_call(
        _mlp_body,
        out_shape=[
            jax.ShapeDtypeStruct((_L, 1), jnp.float32),
            jax.ShapeDtypeStruct((_L, 1), jnp.float32),
        ],
    )


_GDN = lax.GatherDimensionNumbers(
    offset_dims=(), collapsed_slice_dims=(0,), start_index_map=(0,))


def _take16(vec, idx):
    return lax.gather(vec, idx.reshape(_L, 1), _GDN, (1,),
                      mode=lax.GatherScatterMode.PROMISE_IN_BOUNDS)


@functools.lru_cache(maxsize=None)
def _gather_call():
    mesh = plsc.VectorSubcoreMesh(core_axis_name="c", subcore_axis_name="s")

    @functools.partial(
        pl.kernel,
        mesh=mesh,
        out_type=[
            jax.ShapeDtypeStruct((BATCH,), jnp.float32),
            jax.ShapeDtypeStruct((BATCH,), jnp.float32),
        ],
        scratch_types=[
            pltpu.VMEM((_BPW,), jnp.int32),
            pltpu.VMEM((_L,), jnp.float32),
            pltpu.VMEM((_L,), jnp.float32),
            pltpu.VMEM((_BPW,), jnp.float32),
            pltpu.VMEM((_BPW,), jnp.float32),
        ],
    )
    def sc_gather(idx_hbm, tl_hbm, ts_hbm, out_l_hbm, out_s_hbm,
                  idx_v, tl_v, ts_v, ol_v, os_v):
        wid = lax.axis_index("s") * _NC + lax.axis_index("c")
        base = wid * _BPW
        pltpu.sync_copy(idx_hbm.at[pl.ds(base, _BPW)], idx_v)
        pltpu.sync_copy(tl_hbm, tl_v)
        pltpu.sync_copy(ts_hbm, ts_v)
        tl = tl_v[...]
        ts = ts_v[...]
        for i in range(_BPW // _L):
            iv = idx_v[pl.ds(i * _L, _L)]
            ol_v[pl.ds(i * _L, _L)] = _take16(tl, iv)
            os_v[pl.ds(i * _L, _L)] = _take16(ts, iv)
        pltpu.sync_copy(ol_v, out_l_hbm.at[pl.ds(base, _BPW)])
        pltpu.sync_copy(os_v, out_s_hbm.at[pl.ds(base, _BPW)])

    return sc_gather


def kernel(e3_idx, table, W1, b1, W2, b2):
    idx = e3_idx.astype(jnp.int32)
    tp = jnp.pad(table, ((0, _L - NUM_E3), (0, 0)))
    tl, ts = _mlp_call()(tp, W1, b1.reshape(1, HID), W2.reshape(1, HID),
                         b2.reshape(1, 1))
    logits, score = _gather_call()(idx, tl.reshape(_L), ts.reshape(_L))
    return logits, score
